# baseline (speedup 1.0000x reference)
import functools

import jax
import jax.numpy as jnp
from jax.experimental import pallas as pl
from jax.experimental.pallas import tpu as pltpu

N_TOKENS = 2048
DIM_UNI = 1024
DIM_HIDDEN = 256
NUM_GENES = 2000
NUM_EXPERTS = 4
NUM_HEADS = 4
DH = DIM_HIDDEN // NUM_HEADS

TB = 256
N_TB = N_TOKENS // TB

_F32 = jnp.float32


_BF16 = jnp.bfloat16


def _mmT(x, w):
    return jax.lax.dot_general(x.astype(w.dtype), w, (((1,), (1,)), ((), ())),
                               preferred_element_type=_F32)


def _gelu(x):
    return 0.5 * x * (1.0 + jax.lax.erf(x * 0.70710678118654752))


def _softplus(x):
    return jnp.where(x > 15.0, x, jnp.log(1.0 + jnp.exp(jnp.minimum(x, 15.0))))


def _sigmoid(x):
    return 1.0 / (1.0 + jnp.exp(-x))


def _ln(x, g, b, eps=1e-5):
    m = jnp.mean(x, axis=-1, keepdims=True)
    v = jnp.mean((x - m) ** 2, axis=-1, keepdims=True)
    return (x - m) * jax.lax.rsqrt(v + eps) * g + b


def _k1_body(pos_ref, bf_ref, vis_ref, imgW_ref, imgb_ref, posW_ref,
             posb_ref, wqkv_ref, bqkv_ref, z_ref, q_ref, k_ref, v_ref):
    xp = 2.0 * jnp.pi * jax.lax.dot_general(
        pos_ref[...], bf_ref[...], (((1,), (0,)), ((), ())),
        preferred_element_type=_F32)
    fe = jnp.concatenate([jnp.sin(xp), jnp.cos(xp)], axis=-1)
    z = (_mmT(vis_ref[...], imgW_ref[...]) + imgb_ref[...]
         + _mmT(fe, posW_ref[...]) + posb_ref[...])
    z_ref[...] = z
    qkv = (_mmT(z, wqkv_ref[...]) + bqkv_ref[...]).astype(_BF16)
    ones = jnp.ones((TB, DH), dtype=_BF16)
    for h in range(NUM_HEADS):
        q_ref[h] = qkv[:, h * DH:(h + 1) * DH] * _BF16(0.125)
        k_ref[h] = qkv[:, DIM_HIDDEN + h * DH:DIM_HIDDEN + (h + 1) * DH]
        v_ref[h] = jnp.concatenate(
            [qkv[:, 2 * DIM_HIDDEN + h * DH:2 * DIM_HIDDEN + (h + 1) * DH],
             ones], axis=-1)


def _k1(vis, pos, p):
    f = pl.pallas_call(
        _k1_body,
        grid=(N_TB,),
        in_specs=[
            pl.BlockSpec((TB, 3), lambda i: (i, 0)),
            pl.BlockSpec((3, 64), lambda i: (0, 0)),
            pl.BlockSpec((TB, DIM_UNI), lambda i: (i, 0)),
            pl.BlockSpec((DIM_HIDDEN, DIM_UNI), lambda i: (0, 0)),
            pl.BlockSpec((1, DIM_HIDDEN), lambda i: (0, 0)),
            pl.BlockSpec((DIM_HIDDEN, 128), lambda i: (0, 0)),
            pl.BlockSpec((1, DIM_HIDDEN), lambda i: (0, 0)),
            pl.BlockSpec((3 * DIM_HIDDEN, DIM_HIDDEN), lambda i: (0, 0)),
            pl.BlockSpec((1, 3 * DIM_HIDDEN), lambda i: (0, 0)),
        ],
        out_specs=[
            pl.BlockSpec((TB, DIM_HIDDEN), lambda i: (i, 0)),
            pl.BlockSpec((NUM_HEADS, TB, DH), lambda i: (0, i, 0)),
            pl.BlockSpec((NUM_HEADS, TB, DH), lambda i: (0, i, 0)),
            pl.BlockSpec((NUM_HEADS, TB, 2 * DH), lambda i: (0, i, 0)),
        ],
        out_shape=[jax.ShapeDtypeStruct((N_TOKENS, DIM_HIDDEN), _F32),
                   jax.ShapeDtypeStruct((NUM_HEADS, N_TOKENS, DH), _BF16),
                   jax.ShapeDtypeStruct((NUM_HEADS, N_TOKENS, DH), _BF16),
                   jax.ShapeDtypeStruct((NUM_HEADS, N_TOKENS, 2 * DH), _BF16)],
        compiler_params=pltpu.CompilerParams(
            dimension_semantics=("parallel",)),
    )
    return f(pos, p['B_fourier'], vis, p['img_W'], p['img_b'][None, :],
             p['pos_W'], p['pos_b'][None, :], p['attn_Wqkv'],
             p['attn_bqkv'][None, :])


def _k2_body(z_ref, grad_ref, q_ref, k_ref, v_ref, wo_ref, bo_ref, lng_ref,
             lnb_ref, rw_ref, rb_ref, z2_ref, probs_ref, eidx_ref, p1_ref):
    heads = []
    for h in range(NUM_HEADS):
        s = jax.lax.dot_general(q_ref[h], k_ref[h], (((1,), (1,)), ((), ())),
                                preferred_element_type=_F32)
        e = jnp.exp(s.astype(_BF16))
        ov = jax.lax.dot_general(e, v_ref[h], (((1,), (0,)), ((), ())),
                                 preferred_element_type=_F32)
        heads.append(ov[:, :DH] * (1.0 / ov[:, DH:DH + 1]))
    o = jnp.concatenate(heads, axis=-1)
    out = _mmT(o.astype(_BF16), wo_ref[...]) + bo_ref[...]
    z2 = _ln(z_ref[...] + out, lng_ref[...], lnb_ref[...])
    z2_ref[...] = z2
    rw = rw_ref[...]
    logits = (jax.lax.dot_general(z2, rw[:, :DIM_HIDDEN],
                                  (((1,), (1,)), ((), ())),
                                  preferred_element_type=_F32)
              + grad_ref[...] * rw[:, DIM_HIDDEN:DIM_HIDDEN + 1].T
              + rb_ref[...])
    mx = jnp.max(logits, axis=-1, keepdims=True)
    ee = jnp.exp(logits - mx)
    probs = ee / jnp.sum(ee, axis=-1, keepdims=True)
    probs_ref[...] = probs
    eidx = jnp.argmax(probs, axis=-1).astype(jnp.int32)
    eidx_ref[...] = eidx[:, None]
    p1_ref[...] = jnp.max(probs, axis=-1, keepdims=True)


def _k2(z, grad, q, k, v, p):
    f = pl.pallas_call(
        _k2_body,
        grid=(N_TB,),
        in_specs=[
            pl.BlockSpec((TB, DIM_HIDDEN), lambda i: (i, 0)),
            pl.BlockSpec((TB, 1), lambda i: (i, 0)),
            pl.BlockSpec((NUM_HEADS, TB, DH), lambda i: (0, i, 0)),
            pl.BlockSpec((NUM_HEADS, N_TOKENS, DH), lambda i: (0, 0, 0)),
            pl.BlockSpec((NUM_HEADS, N_TOKENS, 2 * DH), lambda i: (0, 0, 0)),
            pl.BlockSpec((DIM_HIDDEN, DIM_HIDDEN), lambda i: (0, 0)),
            pl.BlockSpec((1, DIM_HIDDEN), lambda i: (0, 0)),
            pl.BlockSpec((1, DIM_HIDDEN), lambda i: (0, 0)),
            pl.BlockSpec((1, DIM_HIDDEN), lambda i: (0, 0)),
            pl.BlockSpec((NUM_EXPERTS, DIM_HIDDEN + 1), lambda i: (0, 0)),
            pl.BlockSpec((1, NUM_EXPERTS), lambda i: (0, 0)),
        ],
        out_specs=[
            pl.BlockSpec((TB, DIM_HIDDEN), lambda i: (i, 0)),
            pl.BlockSpec((TB, NUM_EXPERTS), lambda i: (i, 0)),
            pl.BlockSpec((TB, 1), lambda i: (i, 0)),
            pl.BlockSpec((TB, 1), lambda i: (i, 0)),
        ],
        out_shape=[
            jax.ShapeDtypeStruct((N_TOKENS, DIM_HIDDEN), _F32),
            jax.ShapeDtypeStruct((N_TOKENS, NUM_EXPERTS), _F32),
            jax.ShapeDtypeStruct((N_TOKENS, 1), jnp.int32),
            jax.ShapeDtypeStruct((N_TOKENS, 1), _F32),
        ],
        compiler_params=pltpu.CompilerParams(
            dimension_semantics=("arbitrary",)),
    )
    return f(z, grad, q, k, v, p['attn_Wo'], p['attn_bo'][None, :],
             p['ln1_g'][None, :], p['ln1_b'][None, :], p['router_W'],
             p['router_b'][None, :])


N_GROUPS = 16
GROUP = N_TOKENS // N_GROUPS
N_FFN_BLOCKS = N_TOKENS // TB + NUM_EXPERTS - 1
N_PAD = N_FFN_BLOCKS * TB


def _k3_body(eidx_ref, slot_ref, icb_ref):
    eidx = eidx_ref[...]
    lane = jax.lax.broadcasted_iota(jnp.int32, (N_TOKENS, NUM_EXPERTS), 1)
    oh = (lane == eidx).astype(_F32)
    oh3 = oh.reshape(N_GROUPS, GROUP, NUM_EXPERTS)
    l128 = (jax.lax.broadcasted_iota(jnp.int32, (GROUP, GROUP), 1)
            < jax.lax.broadcasted_iota(jnp.int32, (GROUP, GROUP), 0)
            ).astype(_F32)
    ranks = []
    for g in range(N_GROUPS):
        ranks.append(jax.lax.dot_general(
            l128, oh3[g], (((1,), (0,)), ((), ())),
            preferred_element_type=_F32))
    ranks3 = jnp.stack(ranks)
    gsum = jnp.sum(oh3, axis=1)
    s16 = (jax.lax.broadcasted_iota(jnp.int32, (N_GROUPS, N_GROUPS), 1)
           < jax.lax.broadcasted_iota(jnp.int32, (N_GROUPS, N_GROUPS), 0)
           ).astype(_F32)
    gcum = jax.lax.dot_general(s16, gsum, (((1,), (0,)), ((), ())),
                               preferred_element_type=_F32)
    rank = (ranks3 + gcum[:, None, :]).reshape(N_TOKENS, NUM_EXPERTS)
    counts = jnp.sum(gsum, axis=0, keepdims=True)
    nb = jnp.floor((counts + _F32(TB - 1)) * _F32(1.0 / TB))
    u4 = (jax.lax.broadcasted_iota(jnp.int32, (NUM_EXPERTS, NUM_EXPERTS), 0)
          < jax.lax.broadcasted_iota(jnp.int32, (NUM_EXPERTS, NUM_EXPERTS), 1)
          ).astype(_F32)
    excl_b = jax.lax.dot_general(nb, u4, (((1,), (0,)), ((), ())),
                                 preferred_element_type=_F32)
    pad_off = excl_b * _F32(TB)
    slot = jnp.sum(oh * (rank + pad_off), axis=1, keepdims=True)
    slot_ref[...] = slot.astype(jnp.int32)
    icb_ref[...] = (excl_b + nb).astype(jnp.int32)


def _k3(eidx):
    f = pl.pallas_call(
        _k3_body,
        out_shape=[
            jax.ShapeDtypeStruct((N_TOKENS, 1), jnp.int32),
            jax.ShapeDtypeStruct((1, NUM_EXPERTS), jnp.int32),
        ],
    )
    return f(eidx)


_SC_WORKERS = 32
_TOK_PER_W = N_TOKENS // _SC_WORKERS


def _sc_mesh():
    from jax.experimental.pallas import tpu_sc as plsc
    return plsc.VectorSubcoreMesh(core_axis_name="c", subcore_axis_name="s")


def _sc_scatter(z2, slot):
    @functools.partial(
        pl.kernel, mesh=_sc_mesh(),
        out_type=jax.ShapeDtypeStruct((N_PAD, DIM_HIDDEN), _F32),
        scratch_types=[
            pltpu.VMEM((_TOK_PER_W,), jnp.int32),
            pltpu.VMEM((_TOK_PER_W, DIM_HIDDEN), _F32),
            pltpu.SemaphoreType.DMA,
        ],
    )
    def k(z2_hbm, slot_hbm, zbuf_hbm, idx_v, rows_v, sem):
        wid = (jax.lax.axis_index("s") * 2 + jax.lax.axis_index("c"))
        base = wid * _TOK_PER_W
        pltpu.sync_copy(slot_hbm.at[pl.ds(base, _TOK_PER_W)], idx_v)
        pltpu.sync_copy(z2_hbm.at[pl.ds(base, _TOK_PER_W)], rows_v)
        pltpu.async_copy(rows_v, zbuf_hbm.at[idx_v], sem).wait()

    return k(z2, slot)


def _sc_gather(ybuf, slot):
    @functools.partial(
        pl.kernel, mesh=_sc_mesh(),
        out_type=jax.ShapeDtypeStruct((N_TOKENS, DIM_HIDDEN), _F32),
        scratch_types=[
            pltpu.VMEM((_TOK_PER_W,), jnp.int32),
            pltpu.VMEM((_TOK_PER_W, DIM_HIDDEN), _F32),
            pltpu.SemaphoreType.DMA,
        ],
    )
    def k(ybuf_hbm, slot_hbm, yg_hbm, idx_v, rows_v, sem):
        wid = (jax.lax.axis_index("s") * 2 + jax.lax.axis_index("c"))
        base = wid * _TOK_PER_W
        pltpu.sync_copy(slot_hbm.at[pl.ds(base, _TOK_PER_W)], idx_v)
        pltpu.async_copy(ybuf_hbm.at[idx_v], rows_v, sem).wait()
        pltpu.sync_copy(rows_v, yg_hbm.at[pl.ds(base, _TOK_PER_W)])

    return k(ybuf, slot)


def _k5ffn_body(icb_ref, zbuf_ref, w1_ref, b1_ref, w2_ref, b2_ref, ybuf_ref):
    b = pl.program_id(0)

    @pl.when(b < icb_ref[3])
    def _():
        z = zbuf_ref[...].astype(_BF16)
        w1 = w1_ref[0].astype(_BF16)
        h = _gelu(jax.lax.dot_general(z, w1, (((1,), (1,)), ((), ())),
                                      preferred_element_type=_F32)
                  + b1_ref[0])
        w2 = w2_ref[0].astype(_BF16)
        eo = jax.lax.dot_general(h.astype(_BF16), w2,
                                 (((1,), (1,)), ((), ())),
                                 preferred_element_type=_F32) + b2_ref[0]
        ybuf_ref[...] = eo


def _expert_of(b, m):
    e = ((m[0] <= b).astype(jnp.int32) + (m[1] <= b).astype(jnp.int32)
         + (m[2] <= b).astype(jnp.int32))
    return e


def _k5ffn(icb, zbuf, p):
    grid_spec = pltpu.PrefetchScalarGridSpec(
        num_scalar_prefetch=1,
        grid=(N_FFN_BLOCKS,),
        in_specs=[
            pl.BlockSpec((TB, DIM_HIDDEN), lambda b, m: (b, 0)),
            pl.BlockSpec((1, 4 * DIM_HIDDEN, DIM_HIDDEN),
                         lambda b, m: (_expert_of(b, m), 0, 0)),
            pl.BlockSpec((1, 1, 4 * DIM_HIDDEN),
                         lambda b, m: (_expert_of(b, m), 0, 0)),
            pl.BlockSpec((1, DIM_HIDDEN, 4 * DIM_HIDDEN),
                         lambda b, m: (_expert_of(b, m), 0, 0)),
            pl.BlockSpec((1, 1, DIM_HIDDEN),
                         lambda b, m: (_expert_of(b, m), 0, 0)),
        ],
        out_specs=pl.BlockSpec((TB, DIM_HIDDEN), lambda b, m: (b, 0)),
    )
    f = pl.pallas_call(
        _k5ffn_body,
        grid_spec=grid_spec,
        out_shape=jax.ShapeDtypeStruct((N_PAD, DIM_HIDDEN), _F32),
    )
    return f(icb, zbuf, p['exp_W1'], p['exp_b1'][:, None, :],
             p['exp_W2'], p['exp_b2'][:, None, :])


def _k5_body(z2_ref, yg_ref, p1_ref, d1w_ref, d1b_ref, dlng_ref, dlnb_ref,
             d2w_ref, d2b_ref, f1w_ref, f1b_ref, f2w_ref, f2b_ref,
             mu_ref, g_ref):
    z3 = z2_ref[...] + p1_ref[...] * yg_ref[...]
    d = _mmT(z3, d1w_ref[...]) + d1b_ref[...]
    d = _gelu(_ln(d, dlng_ref[...], dlnb_ref[...]))
    mu_ref[...] = _softplus(_mmT(d, d2w_ref[...]) + d2b_ref[...])
    fh = _gelu(_mmT(z3, f1w_ref[...]) + f1b_ref[...])
    g_lin = jnp.sum(fh * f2w_ref[...], axis=-1, keepdims=True)
    g_ref[...] = _sigmoid(g_lin + f2b_ref[0, 0])


def _k5(z2, yg, p1, p):
    d2w_even = p['dec2_W'].reshape(NUM_GENES, 2, DIM_HIDDEN)[:, 0, :].astype(_BF16)
    d2b_even = p['dec2_b'].reshape(NUM_GENES, 2)[:, 0]
    f = pl.pallas_call(
        _k5_body,
        grid=(N_TB,),
        in_specs=[
            pl.BlockSpec((TB, DIM_HIDDEN), lambda i: (i, 0)),
            pl.BlockSpec((TB, DIM_HIDDEN), lambda i: (i, 0)),
            pl.BlockSpec((TB, 1), lambda i: (i, 0)),
            pl.BlockSpec((DIM_HIDDEN, DIM_HIDDEN), lambda i: (0, 0)),
            pl.BlockSpec((1, DIM_HIDDEN), lambda i: (0, 0)),
            pl.BlockSpec((1, DIM_HIDDEN), lambda i: (0, 0)),
            pl.BlockSpec((1, DIM_HIDDEN), lambda i: (0, 0)),
            pl.BlockSpec((NUM_GENES, DIM_HIDDEN), lambda i: (0, 0)),
            pl.BlockSpec((1, NUM_GENES), lambda i: (0, 0)),
            pl.BlockSpec((64, DIM_HIDDEN), lambda i: (0, 0)),
            pl.BlockSpec((1, 64), lambda i: (0, 0)),
            pl.BlockSpec((1, 64), lambda i: (0, 0)),
            pl.BlockSpec((1, 1), lambda i: (0, 0)),
        ],
        out_specs=[
            pl.BlockSpec((TB, NUM_GENES), lambda i: (i, 0)),
            pl.BlockSpec((TB, 1), lambda i: (i, 0)),
        ],
        out_shape=[
            jax.ShapeDtypeStruct((N_TOKENS, NUM_GENES), _F32),
            jax.ShapeDtypeStruct((N_TOKENS, 1), _F32),
        ],
        compiler_params=pltpu.CompilerParams(
            dimension_semantics=("parallel",)),
    )
    return f(z2, yg, p1, p['dec1_W'], p['dec1_b'][None, :], p['dec_ln_g'][None, :],
             p['dec_ln_b'][None, :], d2w_even, d2b_even[None, :],
             p['fh1_W'], p['fh1_b'][None, :], p['fh2_W'],
             p['fh2_b'][None, :])


_BF16_WEIGHTS = ('img_W', 'pos_W', 'attn_Wqkv', 'attn_Wo', 'dec1_W', 'fh1_W')


def kernel(vis, pos, grad, params):
    p = dict(params)
    for name in _BF16_WEIGHTS:
        p[name] = p[name].astype(_BF16)
    z, q, k, v = _k1(vis, pos, p)
    z2, probs, eidx, p1 = _k2(z, grad, q, k, v, p)
    slot2d, icb2d = _k3(eidx)
    slot = slot2d.reshape(N_TOKENS)
    icb = icb2d.reshape(NUM_EXPERTS)
    zbuf = _sc_scatter(z2, slot)
    ybuf = _k5ffn(icb, zbuf, p)
    yg = _sc_gather(ybuf, slot)
    mu, g = _k5(z2, yg, p1, p)
    return mu, g, probs


# speedup vs baseline: 1.0658x; 1.0658x over previous
import functools

import jax
import jax.numpy as jnp
from jax.experimental import pallas as pl
from jax.experimental.pallas import tpu as pltpu

N_TOKENS = 2048
DIM_UNI = 1024
DIM_HIDDEN = 256
NUM_GENES = 2000
NUM_EXPERTS = 4
NUM_HEADS = 4
DH = DIM_HIDDEN // NUM_HEADS

TB = 256
N_TB = N_TOKENS // TB

_F32 = jnp.float32


_BF16 = jnp.bfloat16


def _mmT(x, w):
    return jax.lax.dot_general(x.astype(w.dtype), w, (((1,), (1,)), ((), ())),
                               preferred_element_type=_F32)


def _gelu(x):
    return 0.5 * x * (1.0 + jax.lax.erf(x * 0.70710678118654752))


def _softplus(x):
    return jnp.where(x > 15.0, x, jnp.log(1.0 + jnp.exp(jnp.minimum(x, 15.0))))


def _sigmoid(x):
    return 1.0 / (1.0 + jnp.exp(-x))


def _ln(x, g, b, eps=1e-5):
    m = jnp.mean(x, axis=-1, keepdims=True)
    v = jnp.mean((x - m) ** 2, axis=-1, keepdims=True)
    return (x - m) * jax.lax.rsqrt(v + eps) * g + b


def _k1_body(pos_ref, bf_ref, vis_ref, imgW_ref, imgb_ref, posW_ref,
             posb_ref, wqkv_ref, bqkv_ref, z_ref, q_ref, k_ref, v_ref):
    xp = 2.0 * jnp.pi * jax.lax.dot_general(
        pos_ref[...], bf_ref[...], (((1,), (0,)), ((), ())),
        preferred_element_type=_F32)
    fe = jnp.concatenate([jnp.sin(xp), jnp.cos(xp)], axis=-1)
    z = (_mmT(vis_ref[...], imgW_ref[...].astype(_BF16)) + imgb_ref[...]
         + _mmT(fe, posW_ref[...].astype(_BF16)) + posb_ref[...])
    z_ref[...] = z
    qkv = (_mmT(z, wqkv_ref[...].astype(_BF16)) + bqkv_ref[...]).astype(_BF16)
    ones = jnp.ones((TB, DH), dtype=_BF16)
    for h in range(NUM_HEADS):
        q_ref[h] = qkv[:, h * DH:(h + 1) * DH] * _BF16(0.125)
        k_ref[h] = qkv[:, DIM_HIDDEN + h * DH:DIM_HIDDEN + (h + 1) * DH]
        v_ref[h] = jnp.concatenate(
            [qkv[:, 2 * DIM_HIDDEN + h * DH:2 * DIM_HIDDEN + (h + 1) * DH],
             ones], axis=-1)


def _k1(vis, pos, p):
    f = pl.pallas_call(
        _k1_body,
        grid=(N_TB,),
        in_specs=[
            pl.BlockSpec((TB, 3), lambda i: (i, 0)),
            pl.BlockSpec((3, 64), lambda i: (0, 0)),
            pl.BlockSpec((TB, DIM_UNI), lambda i: (i, 0)),
            pl.BlockSpec((DIM_HIDDEN, DIM_UNI), lambda i: (0, 0)),
            pl.BlockSpec((1, DIM_HIDDEN), lambda i: (0, 0)),
            pl.BlockSpec((DIM_HIDDEN, 128), lambda i: (0, 0)),
            pl.BlockSpec((1, DIM_HIDDEN), lambda i: (0, 0)),
            pl.BlockSpec((3 * DIM_HIDDEN, DIM_HIDDEN), lambda i: (0, 0)),
            pl.BlockSpec((1, 3 * DIM_HIDDEN), lambda i: (0, 0)),
        ],
        out_specs=[
            pl.BlockSpec((TB, DIM_HIDDEN), lambda i: (i, 0)),
            pl.BlockSpec((NUM_HEADS, TB, DH), lambda i: (0, i, 0)),
            pl.BlockSpec((NUM_HEADS, TB, DH), lambda i: (0, i, 0)),
            pl.BlockSpec((NUM_HEADS, TB, 2 * DH), lambda i: (0, i, 0)),
        ],
        out_shape=[jax.ShapeDtypeStruct((N_TOKENS, DIM_HIDDEN), _F32),
                   jax.ShapeDtypeStruct((NUM_HEADS, N_TOKENS, DH), _BF16),
                   jax.ShapeDtypeStruct((NUM_HEADS, N_TOKENS, DH), _BF16),
                   jax.ShapeDtypeStruct((NUM_HEADS, N_TOKENS, 2 * DH), _BF16)],
        compiler_params=pltpu.CompilerParams(
            dimension_semantics=("parallel",)),
    )
    return f(pos, p['B_fourier'], vis, p['img_W'], p['img_b'][None, :],
             p['pos_W'], p['pos_b'][None, :], p['attn_Wqkv'],
             p['attn_bqkv'][None, :])


def _k2_body(z_ref, grad_ref, q_ref, k_ref, v_ref, wo_ref, bo_ref, lng_ref,
             lnb_ref, rw_ref, rb_ref, z2_ref, probs_ref, p1_ref,
             slot_ref, icb_ref, eacc_ref):
    i = pl.program_id(0)
    heads = []
    for h in range(NUM_HEADS):
        s = jax.lax.dot_general(q_ref[h], k_ref[h], (((1,), (1,)), ((), ())),
                                preferred_element_type=_F32)
        e = jnp.exp(s.astype(_BF16))
        ov = jax.lax.dot_general(e, v_ref[h], (((1,), (0,)), ((), ())),
                                 preferred_element_type=_F32)
        heads.append(ov[:, :DH] * (1.0 / ov[:, DH:DH + 1]))
    o = jnp.concatenate(heads, axis=-1)
    out = _mmT(o.astype(_BF16), wo_ref[...].astype(_BF16)) + bo_ref[...]
    z2 = _ln(z_ref[...] + out, lng_ref[...], lnb_ref[...])
    z2_ref[...] = z2
    rw = rw_ref[...]
    logits = (jax.lax.dot_general(z2, rw[:, :DIM_HIDDEN],
                                  (((1,), (1,)), ((), ())),
                                  preferred_element_type=_F32)
              + grad_ref[...] * rw[:, DIM_HIDDEN:DIM_HIDDEN + 1].T
              + rb_ref[...])
    mx = jnp.max(logits, axis=-1, keepdims=True)
    ee = jnp.exp(logits - mx)
    probs = ee / jnp.sum(ee, axis=-1, keepdims=True)
    probs_ref[...] = probs
    eidx = jnp.argmax(probs, axis=-1).astype(jnp.int32)
    eacc_ref[pl.ds(i * TB, TB), :] = eidx[:, None]
    p1_ref[...] = jnp.max(probs, axis=-1, keepdims=True)

    @pl.when(i == N_TB - 1)
    def _():
        _slots_from_eidx(eacc_ref[...], slot_ref, icb_ref)


def _k2(z, grad, q, k, v, p):
    f = pl.pallas_call(
        _k2_body,
        grid=(N_TB,),
        in_specs=[
            pl.BlockSpec((TB, DIM_HIDDEN), lambda i: (i, 0)),
            pl.BlockSpec((TB, 1), lambda i: (i, 0)),
            pl.BlockSpec((NUM_HEADS, TB, DH), lambda i: (0, i, 0)),
            pl.BlockSpec((NUM_HEADS, N_TOKENS, DH), lambda i: (0, 0, 0)),
            pl.BlockSpec((NUM_HEADS, N_TOKENS, 2 * DH), lambda i: (0, 0, 0)),
            pl.BlockSpec((DIM_HIDDEN, DIM_HIDDEN), lambda i: (0, 0)),
            pl.BlockSpec((1, DIM_HIDDEN), lambda i: (0, 0)),
            pl.BlockSpec((1, DIM_HIDDEN), lambda i: (0, 0)),
            pl.BlockSpec((1, DIM_HIDDEN), lambda i: (0, 0)),
            pl.BlockSpec((NUM_EXPERTS, DIM_HIDDEN + 1), lambda i: (0, 0)),
            pl.BlockSpec((1, NUM_EXPERTS), lambda i: (0, 0)),
        ],
        out_specs=[
            pl.BlockSpec((TB, DIM_HIDDEN), lambda i: (i, 0)),
            pl.BlockSpec((TB, NUM_EXPERTS), lambda i: (i, 0)),
            pl.BlockSpec((TB, 1), lambda i: (i, 0)),
            pl.BlockSpec((N_TOKENS, 1), lambda i: (0, 0)),
            pl.BlockSpec((1, NUM_EXPERTS), lambda i: (0, 0)),
        ],
        out_shape=[
            jax.ShapeDtypeStruct((N_TOKENS, DIM_HIDDEN), _F32),
            jax.ShapeDtypeStruct((N_TOKENS, NUM_EXPERTS), _F32),
            jax.ShapeDtypeStruct((N_TOKENS, 1), _F32),
            jax.ShapeDtypeStruct((N_TOKENS, 1), jnp.int32),
            jax.ShapeDtypeStruct((1, NUM_EXPERTS), jnp.int32),
        ],
        scratch_shapes=[pltpu.VMEM((N_TOKENS, 1), jnp.int32)],
        compiler_params=pltpu.CompilerParams(
            dimension_semantics=("arbitrary",)),
    )
    return f(z, grad, q, k, v, p['attn_Wo'], p['attn_bo'][None, :],
             p['ln1_g'][None, :], p['ln1_b'][None, :], p['router_W'],
             p['router_b'][None, :])


N_GROUPS = 16
GROUP = N_TOKENS // N_GROUPS
N_FFN_BLOCKS = N_TOKENS // TB + NUM_EXPERTS - 1
N_PAD = N_FFN_BLOCKS * TB


def _slots_from_eidx(eidx, slot_ref, icb_ref):
    lane = jax.lax.broadcasted_iota(jnp.int32, (N_TOKENS, NUM_EXPERTS), 1)
    oh = (lane == eidx).astype(_F32)
    oh3 = oh.reshape(N_GROUPS, GROUP, NUM_EXPERTS)
    l128 = (jax.lax.broadcasted_iota(jnp.int32, (GROUP, GROUP), 1)
            < jax.lax.broadcasted_iota(jnp.int32, (GROUP, GROUP), 0)
            ).astype(_F32)
    ranks = []
    for g in range(N_GROUPS):
        ranks.append(jax.lax.dot_general(
            l128, oh3[g], (((1,), (0,)), ((), ())),
            preferred_element_type=_F32))
    ranks3 = jnp.stack(ranks)
    gsum = jnp.sum(oh3, axis=1)
    s16 = (jax.lax.broadcasted_iota(jnp.int32, (N_GROUPS, N_GROUPS), 1)
           < jax.lax.broadcasted_iota(jnp.int32, (N_GROUPS, N_GROUPS), 0)
           ).astype(_F32)
    gcum = jax.lax.dot_general(s16, gsum, (((1,), (0,)), ((), ())),
                               preferred_element_type=_F32)
    rank = (ranks3 + gcum[:, None, :]).reshape(N_TOKENS, NUM_EXPERTS)
    counts = jnp.sum(gsum, axis=0, keepdims=True)
    nb = jnp.floor((counts + _F32(TB - 1)) * _F32(1.0 / TB))
    u4 = (jax.lax.broadcasted_iota(jnp.int32, (NUM_EXPERTS, NUM_EXPERTS), 0)
          < jax.lax.broadcasted_iota(jnp.int32, (NUM_EXPERTS, NUM_EXPERTS), 1)
          ).astype(_F32)
    excl_b = jax.lax.dot_general(nb, u4, (((1,), (0,)), ((), ())),
                                 preferred_element_type=_F32)
    pad_off = excl_b * _F32(TB)
    slot = jnp.sum(oh * (rank + pad_off), axis=1, keepdims=True)
    slot_ref[...] = slot.astype(jnp.int32)
    icb_ref[...] = (excl_b + nb).astype(jnp.int32)


_SC_WORKERS = 32
_TOK_PER_W = N_TOKENS // _SC_WORKERS


def _sc_mesh():
    from jax.experimental.pallas import tpu_sc as plsc
    return plsc.VectorSubcoreMesh(core_axis_name="c", subcore_axis_name="s")


def _sc_scatter(z2, slot):
    @functools.partial(
        pl.kernel, mesh=_sc_mesh(),
        out_type=jax.ShapeDtypeStruct((N_PAD, DIM_HIDDEN), _F32),
        scratch_types=[
            pltpu.VMEM((_TOK_PER_W,), jnp.int32),
            pltpu.VMEM((_TOK_PER_W, DIM_HIDDEN), _F32),
            pltpu.SemaphoreType.DMA,
        ],
    )
    def k(z2_hbm, slot_hbm, zbuf_hbm, idx_v, rows_v, sem):
        wid = (jax.lax.axis_index("s") * 2 + jax.lax.axis_index("c"))
        base = wid * _TOK_PER_W
        pltpu.sync_copy(slot_hbm.at[pl.ds(base, _TOK_PER_W)], idx_v)
        pltpu.sync_copy(z2_hbm.at[pl.ds(base, _TOK_PER_W)], rows_v)
        pltpu.async_copy(rows_v, zbuf_hbm.at[idx_v], sem).wait()

    return k(z2, slot)


def _sc_gather(ybuf, slot):
    @functools.partial(
        pl.kernel, mesh=_sc_mesh(),
        out_type=jax.ShapeDtypeStruct((N_TOKENS, DIM_HIDDEN), _F32),
        scratch_types=[
            pltpu.VMEM((_TOK_PER_W,), jnp.int32),
            pltpu.VMEM((_TOK_PER_W, DIM_HIDDEN), _F32),
            pltpu.SemaphoreType.DMA,
        ],
    )
    def k(ybuf_hbm, slot_hbm, yg_hbm, idx_v, rows_v, sem):
        wid = (jax.lax.axis_index("s") * 2 + jax.lax.axis_index("c"))
        base = wid * _TOK_PER_W
        pltpu.sync_copy(slot_hbm.at[pl.ds(base, _TOK_PER_W)], idx_v)
        pltpu.async_copy(ybuf_hbm.at[idx_v], rows_v, sem).wait()
        pltpu.sync_copy(rows_v, yg_hbm.at[pl.ds(base, _TOK_PER_W)])

    return k(ybuf, slot)


def _k5ffn_body(icb_ref, zbuf_ref, w1_ref, b1_ref, w2_ref, b2_ref, ybuf_ref):
    b = pl.program_id(0)

    @pl.when(b < icb_ref[3])
    def _():
        z = zbuf_ref[...].astype(_BF16)
        w1 = w1_ref[0].astype(_BF16)
        h = _gelu(jax.lax.dot_general(z, w1, (((1,), (1,)), ((), ())),
                                      preferred_element_type=_F32)
                  + b1_ref[0])
        w2 = w2_ref[0].astype(_BF16)
        eo = jax.lax.dot_general(h.astype(_BF16), w2,
                                 (((1,), (1,)), ((), ())),
                                 preferred_element_type=_F32) + b2_ref[0]
        ybuf_ref[...] = eo


def _expert_of(b, m):
    e = ((m[0] <= b).astype(jnp.int32) + (m[1] <= b).astype(jnp.int32)
         + (m[2] <= b).astype(jnp.int32))
    return e


def _k5ffn(icb, zbuf, p):
    grid_spec = pltpu.PrefetchScalarGridSpec(
        num_scalar_prefetch=1,
        grid=(N_FFN_BLOCKS,),
        in_specs=[
            pl.BlockSpec((TB, DIM_HIDDEN), lambda b, m: (b, 0)),
            pl.BlockSpec((1, 4 * DIM_HIDDEN, DIM_HIDDEN),
                         lambda b, m: (_expert_of(b, m), 0, 0)),
            pl.BlockSpec((1, 1, 4 * DIM_HIDDEN),
                         lambda b, m: (_expert_of(b, m), 0, 0)),
            pl.BlockSpec((1, DIM_HIDDEN, 4 * DIM_HIDDEN),
                         lambda b, m: (_expert_of(b, m), 0, 0)),
            pl.BlockSpec((1, 1, DIM_HIDDEN),
                         lambda b, m: (_expert_of(b, m), 0, 0)),
        ],
        out_specs=pl.BlockSpec((TB, DIM_HIDDEN), lambda b, m: (b, 0)),
    )
    f = pl.pallas_call(
        _k5ffn_body,
        grid_spec=grid_spec,
        out_shape=jax.ShapeDtypeStruct((N_PAD, DIM_HIDDEN), _F32),
    )
    return f(icb, zbuf, p['exp_W1'], p['exp_b1'][:, None, :],
             p['exp_W2'], p['exp_b2'][:, None, :])


def _k5_body(z2_ref, yg_ref, p1_ref, d1w_ref, d1b_ref, dlng_ref, dlnb_ref,
             d2w_ref, d2b_ref, f1w_ref, f1b_ref, f2w_ref, f2b_ref,
             mu_ref, g_ref):
    z3 = z2_ref[...] + p1_ref[...] * yg_ref[...]
    d = _mmT(z3, d1w_ref[...].astype(_BF16)) + d1b_ref[...]
    d = _gelu(_ln(d, dlng_ref[...], dlnb_ref[...]))
    mu_ref[...] = _softplus(_mmT(d, d2w_ref[...]) + d2b_ref[...])
    fh = _gelu(_mmT(z3, f1w_ref[...].astype(_BF16)) + f1b_ref[...])
    g_lin = jnp.sum(fh * f2w_ref[...], axis=-1, keepdims=True)
    g_ref[...] = _sigmoid(g_lin + f2b_ref[0, 0])


def _k5(z2, yg, p1, p):
    d2w_even = p['dec2_W'].reshape(NUM_GENES, 2, DIM_HIDDEN)[:, 0, :].astype(_BF16)
    d2b_even = p['dec2_b'].reshape(NUM_GENES, 2)[:, 0]
    f = pl.pallas_call(
        _k5_body,
        grid=(N_TB,),
        in_specs=[
            pl.BlockSpec((TB, DIM_HIDDEN), lambda i: (i, 0)),
            pl.BlockSpec((TB, DIM_HIDDEN), lambda i: (i, 0)),
            pl.BlockSpec((TB, 1), lambda i: (i, 0)),
            pl.BlockSpec((DIM_HIDDEN, DIM_HIDDEN), lambda i: (0, 0)),
            pl.BlockSpec((1, DIM_HIDDEN), lambda i: (0, 0)),
            pl.BlockSpec((1, DIM_HIDDEN), lambda i: (0, 0)),
            pl.BlockSpec((1, DIM_HIDDEN), lambda i: (0, 0)),
            pl.BlockSpec((NUM_GENES, DIM_HIDDEN), lambda i: (0, 0)),
            pl.BlockSpec((1, NUM_GENES), lambda i: (0, 0)),
            pl.BlockSpec((64, DIM_HIDDEN), lambda i: (0, 0)),
            pl.BlockSpec((1, 64), lambda i: (0, 0)),
            pl.BlockSpec((1, 64), lambda i: (0, 0)),
            pl.BlockSpec((1, 1), lambda i: (0, 0)),
        ],
        out_specs=[
            pl.BlockSpec((TB, NUM_GENES), lambda i: (i, 0)),
            pl.BlockSpec((TB, 1), lambda i: (i, 0)),
        ],
        out_shape=[
            jax.ShapeDtypeStruct((N_TOKENS, NUM_GENES), _F32),
            jax.ShapeDtypeStruct((N_TOKENS, 1), _F32),
        ],
        compiler_params=pltpu.CompilerParams(
            dimension_semantics=("parallel",)),
    )
    return f(z2, yg, p1, p['dec1_W'], p['dec1_b'][None, :], p['dec_ln_g'][None, :],
             p['dec_ln_b'][None, :], d2w_even, d2b_even[None, :],
             p['fh1_W'], p['fh1_b'][None, :], p['fh2_W'],
             p['fh2_b'][None, :])


def kernel(vis, pos, grad, params):
    p = params
    z, q, k, v = _k1(vis, pos, p)
    z2, probs, p1, slot2d, icb2d = _k2(z, grad, q, k, v, p)
    slot = slot2d.reshape(N_TOKENS)
    icb = icb2d.reshape(NUM_EXPERTS)
    zbuf = _sc_scatter(z2, slot)
    ybuf = _k5ffn(icb, zbuf, p)
    yg = _sc_gather(ybuf, slot)
    mu, g = _k5(z2, yg, p1, p)
    return mu, g, probs


# speedup vs baseline: 1.0848x; 1.0179x over previous
import functools

import jax
import jax.numpy as jnp
from jax.experimental import pallas as pl
from jax.experimental.pallas import tpu as pltpu

N_TOKENS = 2048
DIM_UNI = 1024
DIM_HIDDEN = 256
NUM_GENES = 2000
NUM_EXPERTS = 4
NUM_HEADS = 4
DH = DIM_HIDDEN // NUM_HEADS

TB = 256
N_TB = N_TOKENS // TB

_F32 = jnp.float32


_BF16 = jnp.bfloat16


def _mmT(x, w):
    return jax.lax.dot_general(x.astype(w.dtype), w, (((1,), (1,)), ((), ())),
                               preferred_element_type=_F32)


def _gelu(x):
    return 0.5 * x * (1.0 + jax.lax.erf(x * 0.70710678118654752))


def _softplus(x):
    return jnp.where(x > 15.0, x, jnp.log(1.0 + jnp.exp(jnp.minimum(x, 15.0))))


def _sigmoid(x):
    return 1.0 / (1.0 + jnp.exp(-x))


def _ln(x, g, b, eps=1e-5):
    m = jnp.mean(x, axis=-1, keepdims=True)
    v = jnp.mean((x - m) ** 2, axis=-1, keepdims=True)
    return (x - m) * jax.lax.rsqrt(v + eps) * g + b


def _k12_body(pos_ref, bf_ref, vis_ref, imgW_ref, imgb_ref, posW_ref,
              posb_ref, wqkv_ref, bqkv_ref, grad_ref, wo_ref, bo_ref,
              lng_ref, lnb_ref, rw_ref, rb_ref,
              z2_ref, z2b_ref, probs_ref, p1_ref, slot_ref, icb_ref,
              zs, qs, ks, vs, eacc_ref):
    i = pl.program_id(0)

    @pl.when(i < N_TB)
    def _encode():
        xp = 2.0 * jnp.pi * jax.lax.dot_general(
            pos_ref[...], bf_ref[...], (((1,), (0,)), ((), ())),
            preferred_element_type=_F32)
        fe = jnp.concatenate([jnp.sin(xp), jnp.cos(xp)], axis=-1)
        z = (_mmT(vis_ref[...], imgW_ref[...].astype(_BF16)) + imgb_ref[...]
             + _mmT(fe, posW_ref[...].astype(_BF16)) + posb_ref[...])
        zs[pl.ds(i * TB, TB), :] = z
        qkv = (_mmT(z, wqkv_ref[...].astype(_BF16))
               + bqkv_ref[...]).astype(_BF16)
        ones = jnp.ones((TB, DH), dtype=_BF16)
        for h in range(NUM_HEADS):
            qs[h, pl.ds(i * TB, TB), :] = (
                qkv[:, h * DH:(h + 1) * DH] * _BF16(0.125))
            ks[h, pl.ds(i * TB, TB), :] = (
                qkv[:, DIM_HIDDEN + h * DH:DIM_HIDDEN + (h + 1) * DH])
            vs[h, pl.ds(i * TB, TB), :] = jnp.concatenate(
                [qkv[:, 2 * DIM_HIDDEN + h * DH:
                     2 * DIM_HIDDEN + (h + 1) * DH], ones], axis=-1)

    @pl.when(i >= N_TB)
    def _attend():
        j = i - N_TB
        heads = []
        for h in range(NUM_HEADS):
            q = qs[h, pl.ds(j * TB, TB), :]
            s = jax.lax.dot_general(q, ks[h], (((1,), (1,)), ((), ())),
                                    preferred_element_type=_F32)
            e = jnp.exp(s.astype(_BF16))
            ov = jax.lax.dot_general(e, vs[h], (((1,), (0,)), ((), ())),
                                     preferred_element_type=_F32)
            heads.append(ov[:, :DH] * (1.0 / ov[:, DH:DH + 1]))
        o = jnp.concatenate(heads, axis=-1)
        out = _mmT(o.astype(_BF16), wo_ref[...].astype(_BF16)) + bo_ref[...]
        z2 = _ln(zs[pl.ds(j * TB, TB), :] + out, lng_ref[...], lnb_ref[...])
        z2_ref[...] = z2
        z2b_ref[...] = z2.astype(_BF16)
        rw = rw_ref[...]
        logits = (jax.lax.dot_general(z2, rw[:, :DIM_HIDDEN],
                                      (((1,), (1,)), ((), ())),
                                      preferred_element_type=_F32)
                  + grad_ref[...] * rw[:, DIM_HIDDEN:DIM_HIDDEN + 1].T
                  + rb_ref[...])
        mx = jnp.max(logits, axis=-1, keepdims=True)
        ee = jnp.exp(logits - mx)
        probs = ee / jnp.sum(ee, axis=-1, keepdims=True)
        probs_ref[...] = probs
        eidx = jnp.argmax(probs, axis=-1).astype(jnp.int32)
        eacc_ref[pl.ds(j * TB, TB), :] = eidx[:, None]
        p1_ref[...] = jnp.max(probs, axis=-1, keepdims=True)

        @pl.when(i == 2 * N_TB - 1)
        def _slots():
            _slots_from_eidx(eacc_ref[...], slot_ref, icb_ref)


def _k12(vis, pos, grad, p):
    enc_ix = lambda i: (jnp.minimum(i, N_TB - 1), 0)
    att_ix = lambda i: (jnp.maximum(i - N_TB, 0), 0)
    const2 = lambda i: (0, 0)
    f = pl.pallas_call(
        _k12_body,
        grid=(2 * N_TB,),
        in_specs=[
            pl.BlockSpec((TB, 3), enc_ix),
            pl.BlockSpec((3, 64), const2),
            pl.BlockSpec((TB, DIM_UNI), enc_ix),
            pl.BlockSpec((DIM_HIDDEN, DIM_UNI), const2),
            pl.BlockSpec((1, DIM_HIDDEN), const2),
            pl.BlockSpec((DIM_HIDDEN, 128), const2),
            pl.BlockSpec((1, DIM_HIDDEN), const2),
            pl.BlockSpec((3 * DIM_HIDDEN, DIM_HIDDEN), const2),
            pl.BlockSpec((1, 3 * DIM_HIDDEN), const2),
            pl.BlockSpec((TB, 1), att_ix),
            pl.BlockSpec((DIM_HIDDEN, DIM_HIDDEN), const2),
            pl.BlockSpec((1, DIM_HIDDEN), const2),
            pl.BlockSpec((1, DIM_HIDDEN), const2),
            pl.BlockSpec((1, DIM_HIDDEN), const2),
            pl.BlockSpec((NUM_EXPERTS, DIM_HIDDEN + 1), const2),
            pl.BlockSpec((1, NUM_EXPERTS), const2),
        ],
        out_specs=[
            pl.BlockSpec((TB, DIM_HIDDEN), att_ix),
            pl.BlockSpec((TB, DIM_HIDDEN), att_ix),
            pl.BlockSpec((TB, NUM_EXPERTS), att_ix),
            pl.BlockSpec((TB, 1), att_ix),
            pl.BlockSpec((N_TOKENS, 1), const2),
            pl.BlockSpec((1, NUM_EXPERTS), const2),
        ],
        out_shape=[
            jax.ShapeDtypeStruct((N_TOKENS, DIM_HIDDEN), _F32),
            jax.ShapeDtypeStruct((N_TOKENS, DIM_HIDDEN), _BF16),
            jax.ShapeDtypeStruct((N_TOKENS, NUM_EXPERTS), _F32),
            jax.ShapeDtypeStruct((N_TOKENS, 1), _F32),
            jax.ShapeDtypeStruct((N_TOKENS, 1), jnp.int32),
            jax.ShapeDtypeStruct((1, NUM_EXPERTS), jnp.int32),
        ],
        scratch_shapes=[
            pltpu.VMEM((N_TOKENS, DIM_HIDDEN), _F32),
            pltpu.VMEM((NUM_HEADS, N_TOKENS, DH), _BF16),
            pltpu.VMEM((NUM_HEADS, N_TOKENS, DH), _BF16),
            pltpu.VMEM((NUM_HEADS, N_TOKENS, 2 * DH), _BF16),
            pltpu.VMEM((N_TOKENS, 1), jnp.int32),
        ],
        compiler_params=pltpu.CompilerParams(
            dimension_semantics=("arbitrary",)),
    )
    return f(pos, p['B_fourier'], vis, p['img_W'], p['img_b'][None, :],
             p['pos_W'], p['pos_b'][None, :], p['attn_Wqkv'],
             p['attn_bqkv'][None, :], grad, p['attn_Wo'],
             p['attn_bo'][None, :], p['ln1_g'][None, :], p['ln1_b'][None, :],
             p['router_W'], p['router_b'][None, :])


N_GROUPS = 16
GROUP = N_TOKENS // N_GROUPS
N_FFN_BLOCKS = N_TOKENS // TB + NUM_EXPERTS - 1
N_PAD = N_FFN_BLOCKS * TB


def _slots_from_eidx(eidx, slot_ref, icb_ref):
    lane = jax.lax.broadcasted_iota(jnp.int32, (N_TOKENS, NUM_EXPERTS), 1)
    oh = (lane == eidx).astype(_F32)
    oh3 = oh.reshape(N_GROUPS, GROUP, NUM_EXPERTS)
    l128 = (jax.lax.broadcasted_iota(jnp.int32, (GROUP, GROUP), 1)
            < jax.lax.broadcasted_iota(jnp.int32, (GROUP, GROUP), 0)
            ).astype(_F32)
    ranks = []
    for g in range(N_GROUPS):
        ranks.append(jax.lax.dot_general(
            l128, oh3[g], (((1,), (0,)), ((), ())),
            preferred_element_type=_F32))
    ranks3 = jnp.stack(ranks)
    gsum = jnp.sum(oh3, axis=1)
    s16 = (jax.lax.broadcasted_iota(jnp.int32, (N_GROUPS, N_GROUPS), 1)
           < jax.lax.broadcasted_iota(jnp.int32, (N_GROUPS, N_GROUPS), 0)
           ).astype(_F32)
    gcum = jax.lax.dot_general(s16, gsum, (((1,), (0,)), ((), ())),
                               preferred_element_type=_F32)
    rank = (ranks3 + gcum[:, None, :]).reshape(N_TOKENS, NUM_EXPERTS)
    counts = jnp.sum(gsum, axis=0, keepdims=True)
    nb = jnp.floor((counts + _F32(TB - 1)) * _F32(1.0 / TB))
    u4 = (jax.lax.broadcasted_iota(jnp.int32, (NUM_EXPERTS, NUM_EXPERTS), 0)
          < jax.lax.broadcasted_iota(jnp.int32, (NUM_EXPERTS, NUM_EXPERTS), 1)
          ).astype(_F32)
    excl_b = jax.lax.dot_general(nb, u4, (((1,), (0,)), ((), ())),
                                 preferred_element_type=_F32)
    pad_off = excl_b * _F32(TB)
    slot = jnp.sum(oh * (rank + pad_off), axis=1, keepdims=True)
    slot_ref[...] = slot.astype(jnp.int32)
    icb_ref[...] = (excl_b + nb).astype(jnp.int32)


_SC_WORKERS = 32
_TOK_PER_W = N_TOKENS // _SC_WORKERS


def _sc_mesh():
    from jax.experimental.pallas import tpu_sc as plsc
    return plsc.VectorSubcoreMesh(core_axis_name="c", subcore_axis_name="s")


def _sc_scatter(z2, slot):
    @functools.partial(
        pl.kernel, mesh=_sc_mesh(),
        out_type=jax.ShapeDtypeStruct((N_PAD, DIM_HIDDEN), _F32),
        scratch_types=[
            pltpu.VMEM((_TOK_PER_W,), jnp.int32),
            pltpu.VMEM((_TOK_PER_W, DIM_HIDDEN), _F32),
            pltpu.SemaphoreType.DMA,
        ],
    )
    def k(z2_hbm, slot_hbm, zbuf_hbm, idx_v, rows_v, sem):
        wid = (jax.lax.axis_index("s") * 2 + jax.lax.axis_index("c"))
        base = wid * _TOK_PER_W
        pltpu.sync_copy(slot_hbm.at[pl.ds(base, _TOK_PER_W)], idx_v)
        pltpu.sync_copy(z2_hbm.at[pl.ds(base, _TOK_PER_W)], rows_v)
        pltpu.async_copy(rows_v, zbuf_hbm.at[idx_v], sem).wait()

    return k(z2, slot)


def _sc_gather(ybuf, slot):
    @functools.partial(
        pl.kernel, mesh=_sc_mesh(),
        out_type=jax.ShapeDtypeStruct((N_TOKENS, DIM_HIDDEN), _F32),
        scratch_types=[
            pltpu.VMEM((_TOK_PER_W,), jnp.int32),
            pltpu.VMEM((_TOK_PER_W, DIM_HIDDEN), _F32),
            pltpu.SemaphoreType.DMA,
        ],
    )
    def k(ybuf_hbm, slot_hbm, yg_hbm, idx_v, rows_v, sem):
        wid = (jax.lax.axis_index("s") * 2 + jax.lax.axis_index("c"))
        base = wid * _TOK_PER_W
        pltpu.sync_copy(slot_hbm.at[pl.ds(base, _TOK_PER_W)], idx_v)
        pltpu.async_copy(ybuf_hbm.at[idx_v], rows_v, sem).wait()
        pltpu.sync_copy(rows_v, yg_hbm.at[pl.ds(base, _TOK_PER_W)])

    return k(ybuf, slot)


def _k5ffn_body(icb_ref, zbuf_ref, w1_ref, b1_ref, w2_ref, b2_ref, ybuf_ref):
    b = pl.program_id(0)

    @pl.when(b < icb_ref[3])
    def _():
        z = zbuf_ref[...].astype(_BF16)
        w1 = w1_ref[0].astype(_BF16)
        h = _gelu(jax.lax.dot_general(z, w1, (((1,), (1,)), ((), ())),
                                      preferred_element_type=_F32)
                  + b1_ref[0])
        w2 = w2_ref[0].astype(_BF16)
        eo = jax.lax.dot_general(h.astype(_BF16), w2,
                                 (((1,), (1,)), ((), ())),
                                 preferred_element_type=_F32) + b2_ref[0]
        ybuf_ref[...] = eo


def _expert_of(b, m):
    e = ((m[0] <= b).astype(jnp.int32) + (m[1] <= b).astype(jnp.int32)
         + (m[2] <= b).astype(jnp.int32))
    return e


def _k5ffn(icb, zbuf, p):
    grid_spec = pltpu.PrefetchScalarGridSpec(
        num_scalar_prefetch=1,
        grid=(N_FFN_BLOCKS,),
        in_specs=[
            pl.BlockSpec((TB, DIM_HIDDEN), lambda b, m: (b, 0)),
            pl.BlockSpec((1, 4 * DIM_HIDDEN, DIM_HIDDEN),
                         lambda b, m: (_expert_of(b, m), 0, 0)),
            pl.BlockSpec((1, 1, 4 * DIM_HIDDEN),
                         lambda b, m: (_expert_of(b, m), 0, 0)),
            pl.BlockSpec((1, DIM_HIDDEN, 4 * DIM_HIDDEN),
                         lambda b, m: (_expert_of(b, m), 0, 0)),
            pl.BlockSpec((1, 1, DIM_HIDDEN),
                         lambda b, m: (_expert_of(b, m), 0, 0)),
        ],
        out_specs=pl.BlockSpec((TB, DIM_HIDDEN), lambda b, m: (b, 0)),
    )
    f = pl.pallas_call(
        _k5ffn_body,
        grid_spec=grid_spec,
        out_shape=jax.ShapeDtypeStruct((N_PAD, DIM_HIDDEN), _F32),
    )
    return f(icb, zbuf, p['exp_W1'], p['exp_b1'][:, None, :],
             p['exp_W2'], p['exp_b2'][:, None, :])


def _k5_body(z2_ref, yg_ref, p1_ref, d1w_ref, d1b_ref, dlng_ref, dlnb_ref,
             d2w_ref, d2b_ref, f1w_ref, f1b_ref, f2w_ref, f2b_ref,
             mu_ref, g_ref):
    z3 = z2_ref[...] + p1_ref[...] * yg_ref[...]
    d = _mmT(z3, d1w_ref[...].astype(_BF16)) + d1b_ref[...]
    d = _gelu(_ln(d, dlng_ref[...], dlnb_ref[...]))
    mu_ref[...] = _softplus(_mmT(d, d2w_ref[...]) + d2b_ref[...])
    fh = _gelu(_mmT(z3, f1w_ref[...].astype(_BF16)) + f1b_ref[...])
    g_lin = jnp.sum(fh * f2w_ref[...], axis=-1, keepdims=True)
    g_ref[...] = _sigmoid(g_lin + f2b_ref[0, 0])


def _k5(z2, yg, p1, p):
    d2w_even = p['dec2_W'].reshape(NUM_GENES, 2, DIM_HIDDEN)[:, 0, :].astype(_BF16)
    d2b_even = p['dec2_b'].reshape(NUM_GENES, 2)[:, 0]
    f = pl.pallas_call(
        _k5_body,
        grid=(N_TB,),
        in_specs=[
            pl.BlockSpec((TB, DIM_HIDDEN), lambda i: (i, 0)),
            pl.BlockSpec((TB, DIM_HIDDEN), lambda i: (i, 0)),
            pl.BlockSpec((TB, 1), lambda i: (i, 0)),
            pl.BlockSpec((DIM_HIDDEN, DIM_HIDDEN), lambda i: (0, 0)),
            pl.BlockSpec((1, DIM_HIDDEN), lambda i: (0, 0)),
            pl.BlockSpec((1, DIM_HIDDEN), lambda i: (0, 0)),
            pl.BlockSpec((1, DIM_HIDDEN), lambda i: (0, 0)),
            pl.BlockSpec((NUM_GENES, DIM_HIDDEN), lambda i: (0, 0)),
            pl.BlockSpec((1, NUM_GENES), lambda i: (0, 0)),
            pl.BlockSpec((64, DIM_HIDDEN), lambda i: (0, 0)),
            pl.BlockSpec((1, 64), lambda i: (0, 0)),
            pl.BlockSpec((1, 64), lambda i: (0, 0)),
            pl.BlockSpec((1, 1), lambda i: (0, 0)),
        ],
        out_specs=[
            pl.BlockSpec((TB, NUM_GENES), lambda i: (i, 0)),
            pl.BlockSpec((TB, 1), lambda i: (i, 0)),
        ],
        out_shape=[
            jax.ShapeDtypeStruct((N_TOKENS, NUM_GENES), _F32),
            jax.ShapeDtypeStruct((N_TOKENS, 1), _F32),
        ],
        compiler_params=pltpu.CompilerParams(
            dimension_semantics=("parallel",)),
    )
    return f(z2, yg, p1, p['dec1_W'], p['dec1_b'][None, :], p['dec_ln_g'][None, :],
             p['dec_ln_b'][None, :], d2w_even, d2b_even[None, :],
             p['fh1_W'], p['fh1_b'][None, :], p['fh2_W'],
             p['fh2_b'][None, :])


def kernel(vis, pos, grad, params):
    p = params
    z2, z2b, probs, p1, slot2d, icb2d = _k12(vis, pos, grad, p)
    slot = slot2d.reshape(N_TOKENS)
    icb = icb2d.reshape(NUM_EXPERTS)
    zbuf = _sc_scatter(z2, slot)
    ybuf = _k5ffn(icb, zbuf, p)
    yg = _sc_gather(ybuf, slot)
    mu, g = _k5(z2, yg, p1, p)
    return mu, g, probs


# speedup vs baseline: 1.0869x; 1.0019x over previous
import functools

import jax
import jax.numpy as jnp
from jax.experimental import pallas as pl
from jax.experimental.pallas import tpu as pltpu

N_TOKENS = 2048
DIM_UNI = 1024
DIM_HIDDEN = 256
NUM_GENES = 2000
NUM_EXPERTS = 4
NUM_HEADS = 4
DH = DIM_HIDDEN // NUM_HEADS

TB = 256
N_TB = N_TOKENS // TB

_F32 = jnp.float32


_BF16 = jnp.bfloat16


def _mmT(x, w):
    return jax.lax.dot_general(x.astype(w.dtype), w, (((1,), (1,)), ((), ())),
                               preferred_element_type=_F32)


def _gelu(x):
    return 0.5 * x * (1.0 + jax.lax.erf(x * 0.70710678118654752))


def _softplus(x):
    return jnp.log(1.0 + jnp.exp(x))


def _sigmoid(x):
    return 1.0 / (1.0 + jnp.exp(-x))


def _ln(x, g, b, eps=1e-5):
    m = jnp.mean(x, axis=-1, keepdims=True)
    v = jnp.mean((x - m) ** 2, axis=-1, keepdims=True)
    return (x - m) * jax.lax.rsqrt(v + eps) * g + b


def _k12_body(pos_ref, bf_ref, vis_ref, imgW_ref, imgb_ref, posW_ref,
              posb_ref, wqkv_ref, bqkv_ref, grad_ref, wo_ref, bo_ref,
              lng_ref, lnb_ref, rw_ref, rb_ref,
              z2_ref, probs_ref, p1_ref, slot_ref, icb_ref,
              zs, qs, ks, vs, eacc_ref):
    i = pl.program_id(0)

    @pl.when(i < N_TB)
    def _encode():
        xp = 2.0 * jnp.pi * jax.lax.dot_general(
            pos_ref[...], bf_ref[...], (((1,), (0,)), ((), ())),
            preferred_element_type=_F32)
        fe = jnp.concatenate([jnp.sin(xp), jnp.cos(xp)], axis=-1)
        z = (_mmT(vis_ref[...], imgW_ref[...].astype(_BF16)) + imgb_ref[...]
             + _mmT(fe, posW_ref[...].astype(_BF16)) + posb_ref[...])
        zs[pl.ds(i * TB, TB), :] = z
        qkv = (_mmT(z, wqkv_ref[...].astype(_BF16))
               + bqkv_ref[...]).astype(_BF16)
        ones = jnp.ones((TB, DH), dtype=_BF16)
        for h in range(NUM_HEADS):
            qs[h, pl.ds(i * TB, TB), :] = (
                qkv[:, h * DH:(h + 1) * DH] * _BF16(0.125))
            ks[h, pl.ds(i * TB, TB), :] = (
                qkv[:, DIM_HIDDEN + h * DH:DIM_HIDDEN + (h + 1) * DH])
            vs[h, pl.ds(i * TB, TB), :] = jnp.concatenate(
                [qkv[:, 2 * DIM_HIDDEN + h * DH:
                     2 * DIM_HIDDEN + (h + 1) * DH], ones], axis=-1)

    @pl.when(i >= N_TB)
    def _attend():
        j = i - N_TB
        heads = []
        for h in range(NUM_HEADS):
            q = qs[h, pl.ds(j * TB, TB), :]
            s = jax.lax.dot_general(q, ks[h], (((1,), (1,)), ((), ())),
                                    preferred_element_type=_F32)
            e = jnp.exp(s.astype(_BF16))
            ov = jax.lax.dot_general(e, vs[h], (((1,), (0,)), ((), ())),
                                     preferred_element_type=_F32)
            heads.append(ov[:, :DH] * (1.0 / ov[:, DH:DH + 1]))
        o = jnp.concatenate(heads, axis=-1)
        out = _mmT(o.astype(_BF16), wo_ref[...].astype(_BF16)) + bo_ref[...]
        z2 = _ln(zs[pl.ds(j * TB, TB), :] + out, lng_ref[...], lnb_ref[...])
        z2_ref[...] = z2
        rw = rw_ref[...]
        logits = (jax.lax.dot_general(z2, rw[:, :DIM_HIDDEN],
                                      (((1,), (1,)), ((), ())),
                                      preferred_element_type=_F32)
                  + grad_ref[...] * rw[:, DIM_HIDDEN:DIM_HIDDEN + 1].T
                  + rb_ref[...])
        mx = jnp.max(logits, axis=-1, keepdims=True)
        ee = jnp.exp(logits - mx)
        probs = ee / jnp.sum(ee, axis=-1, keepdims=True)
        probs_ref[...] = probs
        eidx = jnp.argmax(probs, axis=-1).astype(jnp.int32)
        eacc_ref[pl.ds(j * TB, TB), :] = eidx[:, None]
        p1_ref[...] = jnp.max(probs, axis=-1, keepdims=True)

        @pl.when(i == 2 * N_TB - 1)
        def _slots():
            _slots_from_eidx(eacc_ref[...], slot_ref, icb_ref)


def _k12(vis, pos, grad, p):
    enc_ix = lambda i: (jnp.minimum(i, N_TB - 1), 0)
    att_ix = lambda i: (jnp.maximum(i - N_TB, 0), 0)
    const2 = lambda i: (0, 0)
    f = pl.pallas_call(
        _k12_body,
        grid=(2 * N_TB,),
        in_specs=[
            pl.BlockSpec((TB, 3), enc_ix),
            pl.BlockSpec((3, 64), const2),
            pl.BlockSpec((TB, DIM_UNI), enc_ix),
            pl.BlockSpec((DIM_HIDDEN, DIM_UNI), const2),
            pl.BlockSpec((1, DIM_HIDDEN), const2),
            pl.BlockSpec((DIM_HIDDEN, 128), const2),
            pl.BlockSpec((1, DIM_HIDDEN), const2),
            pl.BlockSpec((3 * DIM_HIDDEN, DIM_HIDDEN), const2),
            pl.BlockSpec((1, 3 * DIM_HIDDEN), const2),
            pl.BlockSpec((TB, 1), att_ix),
            pl.BlockSpec((DIM_HIDDEN, DIM_HIDDEN), const2),
            pl.BlockSpec((1, DIM_HIDDEN), const2),
            pl.BlockSpec((1, DIM_HIDDEN), const2),
            pl.BlockSpec((1, DIM_HIDDEN), const2),
            pl.BlockSpec((NUM_EXPERTS, DIM_HIDDEN + 1), const2),
            pl.BlockSpec((1, NUM_EXPERTS), const2),
        ],
        out_specs=[
            pl.BlockSpec((TB, DIM_HIDDEN), att_ix),
            pl.BlockSpec((TB, NUM_EXPERTS), att_ix),
            pl.BlockSpec((TB, 1), att_ix),
            pl.BlockSpec((N_TOKENS, 1), const2),
            pl.BlockSpec((1, NUM_EXPERTS), const2),
        ],
        out_shape=[
            jax.ShapeDtypeStruct((N_TOKENS, DIM_HIDDEN), _F32),
            jax.ShapeDtypeStruct((N_TOKENS, NUM_EXPERTS), _F32),
            jax.ShapeDtypeStruct((N_TOKENS, 1), _F32),
            jax.ShapeDtypeStruct((N_TOKENS, 1), jnp.int32),
            jax.ShapeDtypeStruct((1, NUM_EXPERTS), jnp.int32),
        ],
        scratch_shapes=[
            pltpu.VMEM((N_TOKENS, DIM_HIDDEN), _F32),
            pltpu.VMEM((NUM_HEADS, N_TOKENS, DH), _BF16),
            pltpu.VMEM((NUM_HEADS, N_TOKENS, DH), _BF16),
            pltpu.VMEM((NUM_HEADS, N_TOKENS, 2 * DH), _BF16),
            pltpu.VMEM((N_TOKENS, 1), jnp.int32),
        ],
        compiler_params=pltpu.CompilerParams(
            dimension_semantics=("arbitrary",)),
    )
    return f(pos, p['B_fourier'], vis, p['img_W'], p['img_b'][None, :],
             p['pos_W'], p['pos_b'][None, :], p['attn_Wqkv'],
             p['attn_bqkv'][None, :], grad, p['attn_Wo'],
             p['attn_bo'][None, :], p['ln1_g'][None, :], p['ln1_b'][None, :],
             p['router_W'], p['router_b'][None, :])


N_GROUPS = 16
GROUP = N_TOKENS // N_GROUPS
N_FFN_BLOCKS = N_TOKENS // TB + NUM_EXPERTS - 1
N_PAD = N_FFN_BLOCKS * TB


def _slots_from_eidx(eidx, slot_ref, icb_ref):
    lane = jax.lax.broadcasted_iota(jnp.int32, (N_TOKENS, NUM_EXPERTS), 1)
    oh = (lane == eidx).astype(_F32)
    oh3 = oh.reshape(N_GROUPS, GROUP, NUM_EXPERTS)
    l128 = (jax.lax.broadcasted_iota(jnp.int32, (GROUP, GROUP), 1)
            < jax.lax.broadcasted_iota(jnp.int32, (GROUP, GROUP), 0)
            ).astype(_F32)
    ranks = []
    for g in range(N_GROUPS):
        ranks.append(jax.lax.dot_general(
            l128, oh3[g], (((1,), (0,)), ((), ())),
            preferred_element_type=_F32))
    ranks3 = jnp.stack(ranks)
    gsum = jnp.sum(oh3, axis=1)
    s16 = (jax.lax.broadcasted_iota(jnp.int32, (N_GROUPS, N_GROUPS), 1)
           < jax.lax.broadcasted_iota(jnp.int32, (N_GROUPS, N_GROUPS), 0)
           ).astype(_F32)
    gcum = jax.lax.dot_general(s16, gsum, (((1,), (0,)), ((), ())),
                               preferred_element_type=_F32)
    rank = (ranks3 + gcum[:, None, :]).reshape(N_TOKENS, NUM_EXPERTS)
    counts = jnp.sum(gsum, axis=0, keepdims=True)
    nb = jnp.floor((counts + _F32(TB - 1)) * _F32(1.0 / TB))
    u4 = (jax.lax.broadcasted_iota(jnp.int32, (NUM_EXPERTS, NUM_EXPERTS), 0)
          < jax.lax.broadcasted_iota(jnp.int32, (NUM_EXPERTS, NUM_EXPERTS), 1)
          ).astype(_F32)
    excl_b = jax.lax.dot_general(nb, u4, (((1,), (0,)), ((), ())),
                                 preferred_element_type=_F32)
    pad_off = excl_b * _F32(TB)
    slot = jnp.sum(oh * (rank + pad_off), axis=1, keepdims=True)
    slot_ref[...] = slot.astype(jnp.int32)
    icb_ref[...] = (excl_b + nb).astype(jnp.int32)


_SC_WORKERS = 32
_TOK_PER_W = N_TOKENS // _SC_WORKERS


def _sc_mesh():
    from jax.experimental.pallas import tpu_sc as plsc
    return plsc.VectorSubcoreMesh(core_axis_name="c", subcore_axis_name="s")


def _sc_scatter(z2, slot):
    @functools.partial(
        pl.kernel, mesh=_sc_mesh(),
        out_type=jax.ShapeDtypeStruct((N_PAD, DIM_HIDDEN), _F32),
        scratch_types=[
            pltpu.VMEM((_TOK_PER_W,), jnp.int32),
            pltpu.VMEM((_TOK_PER_W, DIM_HIDDEN), _F32),
            pltpu.SemaphoreType.DMA,
        ],
    )
    def k(z2_hbm, slot_hbm, zbuf_hbm, idx_v, rows_v, sem):
        wid = (jax.lax.axis_index("s") * 2 + jax.lax.axis_index("c"))
        base = wid * _TOK_PER_W
        pltpu.sync_copy(slot_hbm.at[pl.ds(base, _TOK_PER_W)], idx_v)
        pltpu.sync_copy(z2_hbm.at[pl.ds(base, _TOK_PER_W)], rows_v)
        pltpu.async_copy(rows_v, zbuf_hbm.at[idx_v], sem).wait()

    return k(z2, slot)


def _sc_gather(ybuf, slot):
    @functools.partial(
        pl.kernel, mesh=_sc_mesh(),
        out_type=jax.ShapeDtypeStruct((N_TOKENS, DIM_HIDDEN), _F32),
        scratch_types=[
            pltpu.VMEM((_TOK_PER_W,), jnp.int32),
            pltpu.VMEM((_TOK_PER_W, DIM_HIDDEN), _F32),
            pltpu.SemaphoreType.DMA,
        ],
    )
    def k(ybuf_hbm, slot_hbm, yg_hbm, idx_v, rows_v, sem):
        wid = (jax.lax.axis_index("s") * 2 + jax.lax.axis_index("c"))
        base = wid * _TOK_PER_W
        pltpu.sync_copy(slot_hbm.at[pl.ds(base, _TOK_PER_W)], idx_v)
        pltpu.async_copy(ybuf_hbm.at[idx_v], rows_v, sem).wait()
        pltpu.sync_copy(rows_v, yg_hbm.at[pl.ds(base, _TOK_PER_W)])

    return k(ybuf, slot)


def _k5ffn_body(icb_ref, zbuf_ref, w1_ref, b1_ref, w2_ref, b2_ref, ybuf_ref):
    b = pl.program_id(0)

    @pl.when(b < icb_ref[3])
    def _():
        z = zbuf_ref[...].astype(_BF16)
        w1 = w1_ref[0].astype(_BF16)
        h = _gelu((jax.lax.dot_general(z, w1, (((1,), (1,)), ((), ())),
                                       preferred_element_type=_F32)
                   + b1_ref[0]).astype(_BF16))
        w2 = w2_ref[0].astype(_BF16)
        eo = jax.lax.dot_general(h, w2,
                                 (((1,), (1,)), ((), ())),
                                 preferred_element_type=_F32) + b2_ref[0]
        ybuf_ref[...] = eo


def _expert_of(b, m):
    e = ((m[0] <= b).astype(jnp.int32) + (m[1] <= b).astype(jnp.int32)
         + (m[2] <= b).astype(jnp.int32))
    return e


def _k5ffn(icb, zbuf, p):
    grid_spec = pltpu.PrefetchScalarGridSpec(
        num_scalar_prefetch=1,
        grid=(N_FFN_BLOCKS,),
        in_specs=[
            pl.BlockSpec((TB, DIM_HIDDEN), lambda b, m: (b, 0)),
            pl.BlockSpec((1, 4 * DIM_HIDDEN, DIM_HIDDEN),
                         lambda b, m: (_expert_of(b, m), 0, 0)),
            pl.BlockSpec((1, 1, 4 * DIM_HIDDEN),
                         lambda b, m: (_expert_of(b, m), 0, 0)),
            pl.BlockSpec((1, DIM_HIDDEN, 4 * DIM_HIDDEN),
                         lambda b, m: (_expert_of(b, m), 0, 0)),
            pl.BlockSpec((1, 1, DIM_HIDDEN),
                         lambda b, m: (_expert_of(b, m), 0, 0)),
        ],
        out_specs=pl.BlockSpec((TB, DIM_HIDDEN), lambda b, m: (b, 0)),
    )
    f = pl.pallas_call(
        _k5ffn_body,
        grid_spec=grid_spec,
        out_shape=jax.ShapeDtypeStruct((N_PAD, DIM_HIDDEN), _F32),
    )
    return f(icb, zbuf, p['exp_W1'], p['exp_b1'][:, None, :],
             p['exp_W2'], p['exp_b2'][:, None, :])


def _k5_body(z2_ref, yg_ref, p1_ref, d1w_ref, d1b_ref, dlng_ref, dlnb_ref,
             d2w_ref, d2b_ref, f1w_ref, f1b_ref, f2w_ref, f2b_ref,
             mu_ref, g_ref):
    z3 = z2_ref[...] + p1_ref[...] * yg_ref[...]
    d = _mmT(z3, d1w_ref[...].astype(_BF16)) + d1b_ref[...]
    d = _gelu(_ln(d, dlng_ref[...], dlnb_ref[...]).astype(_BF16))
    mu_ref[...] = _softplus(_mmT(d, d2w_ref[...]) + d2b_ref[...])
    fh = _gelu((_mmT(z3, f1w_ref[...].astype(_BF16))
                + f1b_ref[...]).astype(_BF16))
    g_lin = jnp.sum(fh * f2w_ref[...], axis=-1, keepdims=True)
    g_ref[...] = _sigmoid(g_lin + f2b_ref[0, 0])


def _k5(z2, yg, p1, p):
    d2w_even = p['dec2_W'].reshape(NUM_GENES, 2, DIM_HIDDEN)[:, 0, :].astype(_BF16)
    d2b_even = p['dec2_b'].reshape(NUM_GENES, 2)[:, 0]
    f = pl.pallas_call(
        _k5_body,
        grid=(N_TB,),
        in_specs=[
            pl.BlockSpec((TB, DIM_HIDDEN), lambda i: (i, 0)),
            pl.BlockSpec((TB, DIM_HIDDEN), lambda i: (i, 0)),
            pl.BlockSpec((TB, 1), lambda i: (i, 0)),
            pl.BlockSpec((DIM_HIDDEN, DIM_HIDDEN), lambda i: (0, 0)),
            pl.BlockSpec((1, DIM_HIDDEN), lambda i: (0, 0)),
            pl.BlockSpec((1, DIM_HIDDEN), lambda i: (0, 0)),
            pl.BlockSpec((1, DIM_HIDDEN), lambda i: (0, 0)),
            pl.BlockSpec((NUM_GENES, DIM_HIDDEN), lambda i: (0, 0)),
            pl.BlockSpec((1, NUM_GENES), lambda i: (0, 0)),
            pl.BlockSpec((64, DIM_HIDDEN), lambda i: (0, 0)),
            pl.BlockSpec((1, 64), lambda i: (0, 0)),
            pl.BlockSpec((1, 64), lambda i: (0, 0)),
            pl.BlockSpec((1, 1), lambda i: (0, 0)),
        ],
        out_specs=[
            pl.BlockSpec((TB, NUM_GENES), lambda i: (i, 0)),
            pl.BlockSpec((TB, 1), lambda i: (i, 0)),
        ],
        out_shape=[
            jax.ShapeDtypeStruct((N_TOKENS, NUM_GENES), _F32),
            jax.ShapeDtypeStruct((N_TOKENS, 1), _F32),
        ],
        compiler_params=pltpu.CompilerParams(
            dimension_semantics=("parallel",)),
    )
    return f(z2, yg, p1, p['dec1_W'], p['dec1_b'][None, :], p['dec_ln_g'][None, :],
             p['dec_ln_b'][None, :], d2w_even, d2b_even[None, :],
             p['fh1_W'], p['fh1_b'][None, :], p['fh2_W'],
             p['fh2_b'][None, :])


def kernel(vis, pos, grad, params):
    p = params
    z2, probs, p1, slot2d, icb2d = _k12(vis, pos, grad, p)
    slot = slot2d.reshape(N_TOKENS)
    icb = icb2d.reshape(NUM_EXPERTS)
    zbuf = _sc_scatter(z2, slot)
    ybuf = _k5ffn(icb, zbuf, p)
    yg = _sc_gather(ybuf, slot)
    mu, g = _k5(z2, yg, p1, p)
    return mu, g, probs


# speedup vs baseline: 1.1105x; 1.0217x over previous
import functools

import jax
import jax.numpy as jnp
from jax.experimental import pallas as pl
from jax.experimental.pallas import tpu as pltpu

N_TOKENS = 2048
DIM_UNI = 1024
DIM_HIDDEN = 256
NUM_GENES = 2000
NUM_EXPERTS = 4
NUM_HEADS = 4
DH = DIM_HIDDEN // NUM_HEADS

TB = 256
N_TB = N_TOKENS // TB

_F32 = jnp.float32


_BF16 = jnp.bfloat16


def _mmT(x, w):
    return jax.lax.dot_general(x.astype(w.dtype), w, (((1,), (1,)), ((), ())),
                               preferred_element_type=_F32)


def _gelu(x):
    return 0.5 * x * (1.0 + jax.lax.erf(x * 0.70710678118654752))


def _softplus(x):
    return jnp.log(1.0 + jnp.exp(x))


def _sigmoid(x):
    return 1.0 / (1.0 + jnp.exp(-x))


def _ln(x, g, b, eps=1e-5):
    m = jnp.mean(x, axis=-1, keepdims=True)
    v = jnp.mean((x - m) ** 2, axis=-1, keepdims=True)
    return (x - m) * jax.lax.rsqrt(v + eps) * g + b


def _k12_body(pos_ref, bf_ref, vis_ref, imgW_ref, imgb_ref, posW_ref,
              posb_ref, wqkv_ref, bqkv_ref, grad_ref, wo_ref, bo_ref,
              lng_ref, lnb_ref, rw_ref, rb_ref,
              z2_ref, probs_ref, p1_ref, slot_ref, icb_ref,
              zs, qs, ks, vs, eacc_ref):
    i = pl.program_id(0)

    @pl.when(i < N_TB)
    def _encode():
        xp = 2.0 * jnp.pi * jax.lax.dot_general(
            pos_ref[...], bf_ref[...], (((1,), (0,)), ((), ())),
            preferred_element_type=_F32)
        fe = jnp.concatenate([jnp.sin(xp), jnp.cos(xp)], axis=-1)
        z = (_mmT(vis_ref[...], imgW_ref[...].astype(_BF16)) + imgb_ref[...]
             + _mmT(fe, posW_ref[...].astype(_BF16)) + posb_ref[...])
        zs[pl.ds(i * TB, TB), :] = z
        qkv = (_mmT(z, wqkv_ref[...].astype(_BF16))
               + bqkv_ref[...]).astype(_BF16)
        ones = jnp.ones((TB, DH), dtype=_BF16)
        for h in range(NUM_HEADS):
            qs[h, pl.ds(i * TB, TB), :] = (
                qkv[:, h * DH:(h + 1) * DH] * _BF16(0.125))
            ks[h, pl.ds(i * TB, TB), :] = (
                qkv[:, DIM_HIDDEN + h * DH:DIM_HIDDEN + (h + 1) * DH])
            vs[h, pl.ds(i * TB, TB), :] = jnp.concatenate(
                [qkv[:, 2 * DIM_HIDDEN + h * DH:
                     2 * DIM_HIDDEN + (h + 1) * DH], ones], axis=-1)

    @pl.when(i >= N_TB)
    def _attend():
        j = i - N_TB
        heads = []
        for h in range(NUM_HEADS):
            q = qs[h, pl.ds(j * TB, TB), :]
            s = jax.lax.dot_general(q, ks[h], (((1,), (1,)), ((), ())),
                                    preferred_element_type=_F32)
            e = jnp.exp(s.astype(_BF16))
            ov = jax.lax.dot_general(e, vs[h], (((1,), (0,)), ((), ())),
                                     preferred_element_type=_F32)
            heads.append(ov[:, :DH] * (1.0 / ov[:, DH:DH + 1]))
        o = jnp.concatenate(heads, axis=-1)
        out = _mmT(o.astype(_BF16), wo_ref[...].astype(_BF16)) + bo_ref[...]
        z2 = _ln(zs[pl.ds(j * TB, TB), :] + out, lng_ref[...], lnb_ref[...])
        z2_ref[...] = z2
        rw = rw_ref[...]
        logits = (jax.lax.dot_general(z2, rw[:, :DIM_HIDDEN],
                                      (((1,), (1,)), ((), ())),
                                      preferred_element_type=_F32)
                  + grad_ref[...] * rw[:, DIM_HIDDEN:DIM_HIDDEN + 1].T
                  + rb_ref[...])
        mx = jnp.max(logits, axis=-1, keepdims=True)
        ee = jnp.exp(logits - mx)
        probs = ee / jnp.sum(ee, axis=-1, keepdims=True)
        probs_ref[...] = probs
        eidx = jnp.argmax(probs, axis=-1).astype(jnp.int32)
        eacc_ref[pl.ds(j * TB, TB), :] = eidx[:, None]
        p1_ref[...] = jnp.max(probs, axis=-1, keepdims=True)

        @pl.when(i == 2 * N_TB - 1)
        def _slots():
            _slots_from_eidx(eacc_ref[...], slot_ref, icb_ref)


def _k12(vis, pos, grad, p):
    enc_ix = lambda i: (jnp.minimum(i, N_TB - 1), 0)
    att_ix = lambda i: (jnp.maximum(i - N_TB, 0), 0)
    const2 = lambda i: (0, 0)
    f = pl.pallas_call(
        _k12_body,
        grid=(2 * N_TB,),
        in_specs=[
            pl.BlockSpec((TB, 3), enc_ix),
            pl.BlockSpec((3, 64), const2),
            pl.BlockSpec((TB, DIM_UNI), enc_ix),
            pl.BlockSpec((DIM_HIDDEN, DIM_UNI), const2),
            pl.BlockSpec((1, DIM_HIDDEN), const2),
            pl.BlockSpec((DIM_HIDDEN, 128), const2),
            pl.BlockSpec((1, DIM_HIDDEN), const2),
            pl.BlockSpec((3 * DIM_HIDDEN, DIM_HIDDEN), const2),
            pl.BlockSpec((1, 3 * DIM_HIDDEN), const2),
            pl.BlockSpec((TB, 1), att_ix),
            pl.BlockSpec((DIM_HIDDEN, DIM_HIDDEN), const2),
            pl.BlockSpec((1, DIM_HIDDEN), const2),
            pl.BlockSpec((1, DIM_HIDDEN), const2),
            pl.BlockSpec((1, DIM_HIDDEN), const2),
            pl.BlockSpec((NUM_EXPERTS, DIM_HIDDEN + 1), const2),
            pl.BlockSpec((1, NUM_EXPERTS), const2),
        ],
        out_specs=[
            pl.BlockSpec((TB, DIM_HIDDEN), att_ix),
            pl.BlockSpec((TB, NUM_EXPERTS), att_ix),
            pl.BlockSpec((TB, 1), att_ix),
            pl.BlockSpec((N_TOKENS, 1), const2),
            pl.BlockSpec((1, NUM_EXPERTS), const2),
        ],
        out_shape=[
            jax.ShapeDtypeStruct((N_TOKENS, DIM_HIDDEN), _F32),
            jax.ShapeDtypeStruct((N_TOKENS, NUM_EXPERTS), _F32),
            jax.ShapeDtypeStruct((N_TOKENS, 1), _F32),
            jax.ShapeDtypeStruct((N_TOKENS, 1), jnp.int32),
            jax.ShapeDtypeStruct((1, NUM_EXPERTS), jnp.int32),
        ],
        scratch_shapes=[
            pltpu.VMEM((N_TOKENS, DIM_HIDDEN), _F32),
            pltpu.VMEM((NUM_HEADS, N_TOKENS, DH), _BF16),
            pltpu.VMEM((NUM_HEADS, N_TOKENS, DH), _BF16),
            pltpu.VMEM((NUM_HEADS, N_TOKENS, 2 * DH), _BF16),
            pltpu.VMEM((N_TOKENS, 1), jnp.int32),
        ],
        compiler_params=pltpu.CompilerParams(
            dimension_semantics=("arbitrary",)),
    )
    return f(pos, p['B_fourier'], vis, p['img_W'], p['img_b'][None, :],
             p['pos_W'], p['pos_b'][None, :], p['attn_Wqkv'],
             p['attn_bqkv'][None, :], grad, p['attn_Wo'],
             p['attn_bo'][None, :], p['ln1_g'][None, :], p['ln1_b'][None, :],
             p['router_W'], p['router_b'][None, :])


N_GROUPS = 16
GROUP = N_TOKENS // N_GROUPS
N_FFN_BLOCKS = N_TOKENS // TB + NUM_EXPERTS - 1
N_PAD = N_FFN_BLOCKS * TB


def _slots_from_eidx(eidx, slot_ref, icb_ref):
    lane = jax.lax.broadcasted_iota(jnp.int32, (N_TOKENS, NUM_EXPERTS), 1)
    oh = (lane == eidx).astype(_F32)
    oh3 = oh.reshape(N_GROUPS, GROUP, NUM_EXPERTS)
    l128 = (jax.lax.broadcasted_iota(jnp.int32, (GROUP, GROUP), 1)
            < jax.lax.broadcasted_iota(jnp.int32, (GROUP, GROUP), 0)
            ).astype(_F32)
    ranks = []
    for g in range(N_GROUPS):
        ranks.append(jax.lax.dot_general(
            l128, oh3[g], (((1,), (0,)), ((), ())),
            preferred_element_type=_F32))
    ranks3 = jnp.stack(ranks)
    gsum = jnp.sum(oh3, axis=1)
    s16 = (jax.lax.broadcasted_iota(jnp.int32, (N_GROUPS, N_GROUPS), 1)
           < jax.lax.broadcasted_iota(jnp.int32, (N_GROUPS, N_GROUPS), 0)
           ).astype(_F32)
    gcum = jax.lax.dot_general(s16, gsum, (((1,), (0,)), ((), ())),
                               preferred_element_type=_F32)
    rank = (ranks3 + gcum[:, None, :]).reshape(N_TOKENS, NUM_EXPERTS)
    counts = jnp.sum(gsum, axis=0, keepdims=True)
    nb = jnp.floor((counts + _F32(TB - 1)) * _F32(1.0 / TB))
    u4 = (jax.lax.broadcasted_iota(jnp.int32, (NUM_EXPERTS, NUM_EXPERTS), 0)
          < jax.lax.broadcasted_iota(jnp.int32, (NUM_EXPERTS, NUM_EXPERTS), 1)
          ).astype(_F32)
    excl_b = jax.lax.dot_general(nb, u4, (((1,), (0,)), ((), ())),
                                 preferred_element_type=_F32)
    pad_off = excl_b * _F32(TB)
    slot = jnp.sum(oh * (rank + pad_off), axis=1, keepdims=True)
    slot_ref[...] = slot.astype(jnp.int32)
    icb_ref[...] = (excl_b + nb).astype(jnp.int32)


_SC_WORKERS = 32
_TOK_PER_W = N_TOKENS // _SC_WORKERS


def _sc_mesh():
    from jax.experimental.pallas import tpu_sc as plsc
    return plsc.VectorSubcoreMesh(core_axis_name="c", subcore_axis_name="s")


def _sc_scatter(z2, slot):
    @functools.partial(
        pl.kernel, mesh=_sc_mesh(),
        out_type=jax.ShapeDtypeStruct((N_PAD, DIM_HIDDEN), _F32),
        scratch_types=[
            pltpu.VMEM((_TOK_PER_W,), jnp.int32),
            pltpu.VMEM((_TOK_PER_W, DIM_HIDDEN), _F32),
            pltpu.SemaphoreType.DMA,
        ],
    )
    def k(z2_hbm, slot_hbm, zbuf_hbm, idx_v, rows_v, sem):
        wid = (jax.lax.axis_index("s") * 2 + jax.lax.axis_index("c"))
        base = wid * _TOK_PER_W
        pltpu.sync_copy(slot_hbm.at[pl.ds(base, _TOK_PER_W)], idx_v)
        pltpu.sync_copy(z2_hbm.at[pl.ds(base, _TOK_PER_W)], rows_v)
        pltpu.async_copy(rows_v, zbuf_hbm.at[idx_v], sem).wait()

    return k(z2, slot)


def _sc_gather(ybuf, slot):
    @functools.partial(
        pl.kernel, mesh=_sc_mesh(),
        out_type=jax.ShapeDtypeStruct((N_TOKENS, DIM_HIDDEN), _F32),
        scratch_types=[
            pltpu.VMEM((_TOK_PER_W,), jnp.int32),
            pltpu.VMEM((_TOK_PER_W, DIM_HIDDEN), _F32),
            pltpu.SemaphoreType.DMA,
        ],
    )
    def k(ybuf_hbm, slot_hbm, yg_hbm, idx_v, rows_v, sem):
        wid = (jax.lax.axis_index("s") * 2 + jax.lax.axis_index("c"))
        base = wid * _TOK_PER_W
        pltpu.sync_copy(slot_hbm.at[pl.ds(base, _TOK_PER_W)], idx_v)
        pltpu.async_copy(ybuf_hbm.at[idx_v], rows_v, sem).wait()
        pltpu.sync_copy(rows_v, yg_hbm.at[pl.ds(base, _TOK_PER_W)])

    return k(ybuf, slot)


def _k5ffn_body(icb_ref, zbuf_ref, w1_ref, b1_ref, w2_ref, b2_ref, ybuf_ref):
    b = pl.program_id(0)

    @pl.when(b < icb_ref[3])
    def _():
        z = zbuf_ref[...].astype(_BF16)
        w1 = w1_ref[0].astype(_BF16)
        h = _gelu((jax.lax.dot_general(z, w1, (((1,), (1,)), ((), ())),
                                       preferred_element_type=_F32)
                   + b1_ref[0]).astype(_BF16))
        w2 = w2_ref[0].astype(_BF16)
        eo = jax.lax.dot_general(h, w2,
                                 (((1,), (1,)), ((), ())),
                                 preferred_element_type=_F32) + b2_ref[0]
        ybuf_ref[...] = eo


def _expert_of(b, m):
    e = ((m[0] <= b).astype(jnp.int32) + (m[1] <= b).astype(jnp.int32)
         + (m[2] <= b).astype(jnp.int32))
    return e


def _k5ffn(icb, zbuf, p):
    grid_spec = pltpu.PrefetchScalarGridSpec(
        num_scalar_prefetch=1,
        grid=(N_FFN_BLOCKS,),
        in_specs=[
            pl.BlockSpec((TB, DIM_HIDDEN), lambda b, m: (b, 0)),
            pl.BlockSpec((1, 4 * DIM_HIDDEN, DIM_HIDDEN),
                         lambda b, m: (_expert_of(b, m), 0, 0)),
            pl.BlockSpec((1, 1, 4 * DIM_HIDDEN),
                         lambda b, m: (_expert_of(b, m), 0, 0)),
            pl.BlockSpec((1, DIM_HIDDEN, 4 * DIM_HIDDEN),
                         lambda b, m: (_expert_of(b, m), 0, 0)),
            pl.BlockSpec((1, 1, DIM_HIDDEN),
                         lambda b, m: (_expert_of(b, m), 0, 0)),
        ],
        out_specs=pl.BlockSpec((TB, DIM_HIDDEN), lambda b, m: (b, 0)),
    )
    f = pl.pallas_call(
        _k5ffn_body,
        grid_spec=grid_spec,
        out_shape=jax.ShapeDtypeStruct((N_PAD, DIM_HIDDEN), _F32),
        compiler_params=pltpu.CompilerParams(
            dimension_semantics=("parallel",)),
    )
    return f(icb, zbuf, p['exp_W1'], p['exp_b1'][:, None, :],
             p['exp_W2'], p['exp_b2'][:, None, :])


def _k5_body(z2_ref, yg_ref, p1_ref, d1w_ref, d1b_ref, dlng_ref, dlnb_ref,
             d2w_ref, d2b_ref, f1w_ref, f1b_ref, f2w_ref, f2b_ref,
             mu_ref, g_ref):
    z3 = z2_ref[...] + p1_ref[...] * yg_ref[...]
    d = _mmT(z3, d1w_ref[...].astype(_BF16)) + d1b_ref[...]
    d = _gelu(_ln(d, dlng_ref[...], dlnb_ref[...]).astype(_BF16))
    mu_ref[...] = _softplus(_mmT(d, d2w_ref[...]) + d2b_ref[...])
    fh = _gelu((_mmT(z3, f1w_ref[...].astype(_BF16))
                + f1b_ref[...]).astype(_BF16))
    g_lin = jnp.sum(fh * f2w_ref[...], axis=-1, keepdims=True)
    g_ref[...] = _sigmoid(g_lin + f2b_ref[0, 0])


DB = 512


def _k5(z2, yg, p1, p):
    d2w_even = p['dec2_W'].reshape(NUM_GENES, 2, DIM_HIDDEN)[:, 0, :].astype(_BF16)
    d2b_even = p['dec2_b'].reshape(NUM_GENES, 2)[:, 0]
    f = pl.pallas_call(
        _k5_body,
        grid=(N_TOKENS // DB,),
        in_specs=[
            pl.BlockSpec((DB, DIM_HIDDEN), lambda i: (i, 0)),
            pl.BlockSpec((DB, DIM_HIDDEN), lambda i: (i, 0)),
            pl.BlockSpec((DB, 1), lambda i: (i, 0)),
            pl.BlockSpec((DIM_HIDDEN, DIM_HIDDEN), lambda i: (0, 0)),
            pl.BlockSpec((1, DIM_HIDDEN), lambda i: (0, 0)),
            pl.BlockSpec((1, DIM_HIDDEN), lambda i: (0, 0)),
            pl.BlockSpec((1, DIM_HIDDEN), lambda i: (0, 0)),
            pl.BlockSpec((NUM_GENES, DIM_HIDDEN), lambda i: (0, 0)),
            pl.BlockSpec((1, NUM_GENES), lambda i: (0, 0)),
            pl.BlockSpec((64, DIM_HIDDEN), lambda i: (0, 0)),
            pl.BlockSpec((1, 64), lambda i: (0, 0)),
            pl.BlockSpec((1, 64), lambda i: (0, 0)),
            pl.BlockSpec((1, 1), lambda i: (0, 0)),
        ],
        out_specs=[
            pl.BlockSpec((DB, NUM_GENES), lambda i: (i, 0)),
            pl.BlockSpec((DB, 1), lambda i: (i, 0)),
        ],
        out_shape=[
            jax.ShapeDtypeStruct((N_TOKENS, NUM_GENES), _F32),
            jax.ShapeDtypeStruct((N_TOKENS, 1), _F32),
        ],
        compiler_params=pltpu.CompilerParams(
            dimension_semantics=("parallel",)),
    )
    return f(z2, yg, p1, p['dec1_W'], p['dec1_b'][None, :], p['dec_ln_g'][None, :],
             p['dec_ln_b'][None, :], d2w_even, d2b_even[None, :],
             p['fh1_W'], p['fh1_b'][None, :], p['fh2_W'],
             p['fh2_b'][None, :])


def kernel(vis, pos, grad, params):
    p = params
    z2, probs, p1, slot2d, icb2d = _k12(vis, pos, grad, p)
    slot = slot2d.reshape(N_TOKENS)
    icb = icb2d.reshape(NUM_EXPERTS)
    zbuf = _sc_scatter(z2, slot)
    ybuf = _k5ffn(icb, zbuf, p)
    yg = _sc_gather(ybuf, slot)
    mu, g = _k5(z2, yg, p1, p)
    return mu, g, probs
